# baseline (speedup 1.0000x reference)
import jax
import jax.numpy as jnp
from jax.experimental import pallas as pl
from jax.experimental.pallas import tpu as pltpu

R, V = 128, 100000
T = 10
CR = V // T
K = 5

NEG = -3.4e38


def _max_body(xt_hbm, o_ref, *scratch):
    bufs = scratch[:K]
    sems = scratch[K:]

    def issue(t):
        return pltpu.make_async_copy(
            xt_hbm.at[pl.ds(t * CR, CR), :], bufs[t % K], sems[t % K])

    cps = [issue(t) for t in range(K)]
    for cp in cps:
        cp.start()
    acc = jnp.full((R,), NEG, jnp.float32)
    for t in range(T):
        cps[t % K].wait()
        acc = jnp.maximum(acc, jnp.max(bufs[t % K][...], axis=0))
        if t + K < T:
            cps[t % K] = issue(t + K)
            cps[t % K].start()
    o_ref[0, :] = acc


def kernel(X):
    out = pl.pallas_call(
        _max_body,
        in_specs=[pl.BlockSpec(memory_space=pl.ANY)],
        out_specs=pl.BlockSpec(memory_space=pltpu.MemorySpace.VMEM),
        out_shape=jax.ShapeDtypeStruct((1, R), jnp.float32),
        scratch_shapes=(
            [pltpu.VMEM((CR, R), jnp.float32) for _ in range(K)]
            + [pltpu.SemaphoreType.DMA for _ in range(K)]
        ),
    )(X.T)
    return out[0]


# speedup vs baseline: 1.0815x; 1.0815x over previous
import jax
import jax.numpy as jnp
from jax.experimental import pallas as pl
from jax.experimental.pallas import tpu as pltpu

R, V = 128, 100000
T = 20
CR = V // T
K = 6

NEG = -3.4e38


def _max_body(xt_hbm, o_ref, *scratch):
    bufs = scratch[:K]
    sems = scratch[K:]

    def issue(t):
        return pltpu.make_async_copy(
            xt_hbm.at[pl.ds(t * CR, CR), :], bufs[t % K], sems[t % K])

    cps = [issue(t) for t in range(K)]
    for cp in cps:
        cp.start()
    acc = jnp.full((R,), NEG, jnp.float32)
    for t in range(T):
        cps[t % K].wait()
        acc = jnp.maximum(acc, jnp.max(bufs[t % K][...], axis=0))
        if t + K < T:
            cps[t % K] = issue(t + K)
            cps[t % K].start()
    o_ref[0, :] = acc


def kernel(X):
    out = pl.pallas_call(
        _max_body,
        in_specs=[pl.BlockSpec(memory_space=pl.ANY)],
        out_specs=pl.BlockSpec(memory_space=pltpu.MemorySpace.VMEM),
        out_shape=jax.ShapeDtypeStruct((1, R), jnp.float32),
        scratch_shapes=(
            [pltpu.VMEM((CR, R), jnp.float32) for _ in range(K)]
            + [pltpu.SemaphoreType.DMA for _ in range(K)]
        ),
    )(X.T)
    return out[0]
